# Initial kernel scaffold; baseline (speedup 1.0000x reference)
import jax
import jax.numpy as jnp
from jax.experimental import pallas as pl


def kernel(features, labels, centers):
    raise NotImplementedError("write your pallas kernel here")


# speedup vs baseline: 20.2682x; 20.2682x over previous
import jax
import jax.numpy as jnp
from jax.experimental import pallas as pl
from jax.experimental.pallas import tpu as pltpu

_N = 16384
_D = 128
_C = 100
_CP = 128
_B = 2048
_K = _N // _B


def _body(lab_ref, feat_ref, cent_ref, out_ref, cs_ref, ni_ref, mc_ref, sc_ref):
    i = pl.program_id(0)
    f = feat_ref[...]
    lab = lab_ref[0, 0, :]
    oh = (lab[:, None] == jax.lax.broadcasted_iota(jnp.int32, (_B, _CP), 1)
          ).astype(jnp.float32)

    @pl.when(i == 0)
    def _init():
        cs_ref[...] = jnp.zeros((_CP, _D), jnp.float32)
        ni_ref[...] = jnp.zeros((1, _CP), jnp.float32)
        sc_ref[0] = 0.0
        sc_ref[1] = 0.0
        sc_ref[2] = 0.0

    @pl.when(i < _K)
    def _phase_a():
        cs_ref[...] += jax.lax.dot_general(
            oh, f, (((0,), (0,)), ((), ())),
            preferred_element_type=jnp.float32,
            precision=jax.lax.Precision.HIGHEST)
        ni_ref[...] += jnp.sum(oh, axis=0, keepdims=True)

    @pl.when(i == _K)
    def _boundary():
        cs = cs_ref[...]
        ni = ni_ref[0, :]
        om = jnp.sum(cs, axis=0) / _N
        ni_safe = jnp.where(ni > 0, ni, 1.0)
        mc = cs / ni_safe[:, None]
        mc_ref[...] = mc
        om_norm = jnp.sqrt(jnp.sum(om * om))
        mcn = jnp.sqrt(jnp.sum(mc * mc, axis=1))
        mcn_safe = jnp.where(mcn > 0, mcn, 1.0)
        d_cls = 1.0 - jnp.sum(mc * om[None, :], axis=1) / mcn_safe * om_norm
        sc_ref[2] = jnp.sum((ni / _N) * d_cls * d_cls)

    @pl.when(i >= _K)
    def _phase_b():
        mc = mc_ref[...]
        g_mc = jax.lax.dot_general(
            oh, mc, (((1,), (0,)), ((), ())),
            preferred_element_type=jnp.float32,
            precision=jax.lax.Precision.HIGHEST)
        g_cb = jax.lax.dot_general(
            oh, cent_ref[...], (((1,), (0,)), ((), ())),
            preferred_element_type=jnp.float32,
            precision=jax.lax.Precision.HIGHEST)
        inv_rn = jax.lax.rsqrt(jnp.sum(f * f, axis=1))
        mcn_g = jnp.sqrt(jnp.sum(g_mc * g_mc, axis=1))
        cbn_g = jnp.sqrt(jnp.sum(g_cb * g_cb, axis=1))
        dw = 1.0 - jnp.sum(f * g_mc, axis=1) * inv_rn * mcn_g
        dc = 1.0 - jnp.sum(f * g_cb, axis=1) * inv_rn * cbn_g
        sc_ref[0] += jnp.sum(dw * dw)
        sc_ref[1] += jnp.sum(dc * dc)

    @pl.when(i == 2 * _K - 1)
    def _final():
        loss = (sc_ref[0] / _N) / sc_ref[2] + sc_ref[1] / _N
        out_ref[...] = jnp.full((1, 1), loss, jnp.float32)


def kernel(features, labels, centers):
    lab3 = labels.reshape(_K, 1, _B)
    cent_p = jnp.pad(centers, ((0, _CP - _C), (0, 0)))
    out = pl.pallas_call(
        _body,
        grid=(2 * _K,),
        in_specs=[
            pl.BlockSpec((1, 1, _B), lambda i: (i % _K, 0, 0)),
            pl.BlockSpec((_B, _D), lambda i: (i % _K, 0)),
            pl.BlockSpec((_CP, _D), lambda i: (0, 0)),
        ],
        out_specs=pl.BlockSpec((1, 1), lambda i: (0, 0)),
        out_shape=jax.ShapeDtypeStruct((1, 1), jnp.float32),
        scratch_shapes=[
            pltpu.VMEM((_CP, _D), jnp.float32),
            pltpu.VMEM((1, _CP), jnp.float32),
            pltpu.VMEM((_CP, _D), jnp.float32),
            pltpu.SMEM((4,), jnp.float32),
        ],
    )(lab3, features, cent_p)
    return out.reshape(())
